# Initial kernel scaffold; baseline (speedup 1.0000x reference)
import functools

import jax
import jax.numpy as jnp
from jax import lax
from jax.experimental import pallas as pl
from jax.experimental.pallas import tpu as pltpu
from jax.experimental.pallas import tpu_sc as plsc

B, L = 4096, 200
V, D_IN, D_OUT = 100, 32, 16
N = B * L
VPAD = 128

NC, NS = 2, 16
NW = NC * NS
ROWS_PER_W = N // (NW * 128)
GROUP = 20
N_GROUPS = ROWS_PER_W // GROUP


def _table_kernel(emb_ref, gamma_ref, beta_ref, w_ref, b_ref, out_ref):
    x = emb_ref[...]
    mean = jnp.mean(x, axis=1, keepdims=True)
    xc = x - mean
    var = jnp.mean(xc * xc, axis=1, keepdims=True)
    xn = xc * lax.rsqrt(var + 1e-5)
    xn = xn * gamma_ref[...] + beta_ref[...]
    out_ref[...] = (
        jnp.dot(xn, w_ref[...], preferred_element_type=jnp.float32)
        + b_ref[...]
    )


def _fuse_table(emb_table, ln_gamma, ln_beta, W, b):
    emb_pad = jnp.zeros((VPAD, D_IN), jnp.float32).at[:V].set(emb_table)
    return pl.pallas_call(
        _table_kernel,
        out_shape=jax.ShapeDtypeStruct((VPAD, D_OUT), jnp.float32),
    )(
        emb_pad,
        ln_gamma.reshape(1, D_IN),
        ln_beta.reshape(1, D_IN),
        W,
        b.reshape(1, D_OUT),
    )


def _sc_gather_body(table_hbm, ids_hbm, out_hbm, idx_v, rows_v, sem):
    wid = lax.axis_index("s") * NC + lax.axis_index("c")
    row0 = wid * ROWS_PER_W
    pltpu.sync_copy(ids_hbm.at[pl.ds(row0, ROWS_PER_W)], idx_v)

    def group_body(g, carry):
        handles = []
        for j in range(GROUP):
            r = g * GROUP + j
            handles.append(
                pltpu.async_copy(table_hbm.at[idx_v.at[r]], rows_v.at[j], sem)
            )
        for h in handles:
            h.wait()
        for j in range(GROUP):
            r = g * GROUP + j
            pltpu.sync_copy(
                rows_v.at[j], out_hbm.at[pl.ds((row0 + r) * 128, 128)]
            )
        return carry

    lax.fori_loop(0, N_GROUPS, group_body, 0)


def _sc_gather(table, ids_flat):
    mesh = plsc.VectorSubcoreMesh(core_axis_name="c", subcore_axis_name="s")
    ids2d = ids_flat.reshape(N // 128, 128)
    run = pl.kernel(
        _sc_gather_body,
        out_type=jax.ShapeDtypeStruct((N, D_OUT), jnp.float32),
        mesh=mesh,
        scratch_types=[
            pltpu.VMEM((ROWS_PER_W, 128), jnp.int32),
            pltpu.VMEM((GROUP, 128, D_OUT), jnp.float32),
            pltpu.SemaphoreType.DMA,
        ],
    )
    return run(table, ids2d)


def kernel(input_ids, emb_table, ln_gamma, ln_beta, W, b):
    table = _fuse_table(emb_table, ln_gamma, ln_beta, W, b)
    ids_flat = input_ids.reshape(N).astype(jnp.int32)
    out = _sc_gather(table, ids_flat)
    return out.reshape(B, L, D_OUT)


# speedup vs baseline: 3.6835x; 3.6835x over previous
import functools

import jax
import jax.numpy as jnp
from jax import lax
from jax.experimental import pallas as pl
from jax.experimental.pallas import tpu as pltpu
from jax.experimental.pallas import tpu_sc as plsc

B, L = 4096, 200
V, D_IN, D_OUT = 100, 32, 16
N = B * L
VPAD = 128

NC, NS = 2, 16
NW = NC * NS
ROWS_PER_W = N // (NW * 128)
GROUP = 20
N_GROUPS = ROWS_PER_W // GROUP


def _table_kernel(emb_ref, gamma_ref, beta_ref, w_ref, b_ref, out_ref):
    x = emb_ref[...]
    mean = jnp.mean(x, axis=1, keepdims=True)
    xc = x - mean
    var = jnp.mean(xc * xc, axis=1, keepdims=True)
    xn = xc * lax.rsqrt(var + 1e-5)
    xn = xn * gamma_ref[...] + beta_ref[...]
    out_ref[...] = (
        jnp.dot(xn, w_ref[...], preferred_element_type=jnp.float32)
        + b_ref[...]
    )


def _fuse_table(emb_table, ln_gamma, ln_beta, W, b):
    emb_pad = jnp.zeros((VPAD, D_IN), jnp.float32).at[:V].set(emb_table)
    return pl.pallas_call(
        _table_kernel,
        out_shape=jax.ShapeDtypeStruct((VPAD, D_OUT), jnp.float32),
    )(
        emb_pad,
        ln_gamma.reshape(1, D_IN),
        ln_beta.reshape(1, D_IN),
        W,
        b.reshape(1, D_OUT),
    )


def _sc_gather_body(table_hbm, ids_hbm, out_hbm, idx_v, rows_v, sem):
    wid = lax.axis_index("s") * NC + lax.axis_index("c")
    row0 = wid * ROWS_PER_W
    pltpu.sync_copy(ids_hbm.at[pl.ds(row0, ROWS_PER_W)], idx_v)

    def group_body(g, carry):
        handles = []
        for j in range(GROUP):
            r = g * GROUP + j
            handles.append(
                pltpu.async_copy(table_hbm.at[idx_v.at[r]], rows_v.at[j], sem)
            )
        for h in handles:
            h.wait()
        for j in range(GROUP):
            r = g * GROUP + j
            pltpu.sync_copy(
                rows_v.at[j], out_hbm.at[pl.ds((row0 + r) * 128, 128)]
            )
        return carry

    lax.fori_loop(0, N_GROUPS, group_body, 0)


def _sc_gather(table, ids_flat):
    mesh = plsc.VectorSubcoreMesh(core_axis_name="c", subcore_axis_name="s")
    ids2d = ids_flat.reshape(N // 128, 128)
    run = pl.kernel(
        _sc_gather_body,
        out_type=jax.ShapeDtypeStruct((N, D_OUT), jnp.float32),
        mesh=mesh,
        scratch_types=[
            pltpu.VMEM((ROWS_PER_W, 128), jnp.int32),
            pltpu.VMEM((GROUP, 128, D_OUT), jnp.float32),
            pltpu.SemaphoreType.DMA,
        ],
        compiler_params=pltpu.CompilerParams(use_tc_tiling_on_sc=False),
    )
    return run(table, ids2d)


def kernel(input_ids, emb_table, ln_gamma, ln_beta, W, b):
    table = _fuse_table(emb_table, ln_gamma, ln_beta, W, b)
    ids_flat = input_ids.reshape(N).astype(jnp.int32)
    out = _sc_gather(table, ids_flat)
    return out.reshape(B, L, D_OUT)


# speedup vs baseline: 3.6904x; 1.0019x over previous
import functools

import jax
import jax.numpy as jnp
from jax import lax
from jax.experimental import pallas as pl
from jax.experimental.pallas import tpu as pltpu
from jax.experimental.pallas import tpu_sc as plsc

B, L = 4096, 200
V, D_IN, D_OUT = 100, 32, 16
N = B * L
VPAD = 128

NC, NS = 2, 16
NW = NC * NS
ROWS_PER_W = N // (NW * 128)
GROUP = 20
N_GROUPS = ROWS_PER_W // GROUP


def _table_kernel(emb_ref, gamma_ref, beta_ref, w_ref, b_ref, out_ref):
    x = emb_ref[...]
    mean = jnp.mean(x, axis=1, keepdims=True)
    xc = x - mean
    var = jnp.mean(xc * xc, axis=1, keepdims=True)
    xn = xc * lax.rsqrt(var + 1e-5)
    xn = xn * gamma_ref[...] + beta_ref[...]
    out_ref[...] = (
        jnp.dot(xn, w_ref[...], preferred_element_type=jnp.float32)
        + b_ref[...]
    )


def _fuse_table(emb_table, ln_gamma, ln_beta, W, b):
    emb_pad = jnp.zeros((VPAD, D_IN), jnp.float32).at[:V].set(emb_table)
    return pl.pallas_call(
        _table_kernel,
        out_shape=jax.ShapeDtypeStruct((VPAD, D_OUT), jnp.float32),
    )(
        emb_pad,
        ln_gamma.reshape(1, D_IN),
        ln_beta.reshape(1, D_IN),
        W,
        b.reshape(1, D_OUT),
    )


def _sc_gather_body(table_hbm, ids_hbm, out_hbm, idx_v, rows_v, sem):
    wid = lax.axis_index("s") * NC + lax.axis_index("c")
    row0 = wid * ROWS_PER_W
    pltpu.sync_copy(ids_hbm.at[pl.ds(row0, ROWS_PER_W)], idx_v)

    def group_body(g, carry):
        handles = []
        for j in range(GROUP):
            r = g * GROUP + j
            handles.append(
                pltpu.async_copy(
                    table_hbm.at[idx_v.at[r]],
                    rows_v.at[pl.ds(j * 128, 128)],
                    sem,
                )
            )
        for h in handles:
            h.wait()
        pltpu.sync_copy(
            rows_v, out_hbm.at[pl.ds((row0 + g * GROUP) * 128, GROUP * 128)]
        )
        return carry

    lax.fori_loop(0, N_GROUPS, group_body, 0)


def _sc_gather(table, ids_flat):
    mesh = plsc.VectorSubcoreMesh(core_axis_name="c", subcore_axis_name="s")
    ids2d = ids_flat.reshape(N // 128, 128)
    run = pl.kernel(
        _sc_gather_body,
        out_type=jax.ShapeDtypeStruct((N, D_OUT), jnp.float32),
        mesh=mesh,
        scratch_types=[
            pltpu.VMEM((ROWS_PER_W, 128), jnp.int32),
            pltpu.VMEM((GROUP * 128, D_OUT), jnp.float32),
            pltpu.SemaphoreType.DMA,
        ],
        compiler_params=pltpu.CompilerParams(use_tc_tiling_on_sc=False),
    )
    return run(table, ids2d)


def kernel(input_ids, emb_table, ln_gamma, ln_beta, W, b):
    table = _fuse_table(emb_table, ln_gamma, ln_beta, W, b)
    ids_flat = input_ids.reshape(N).astype(jnp.int32)
    out = _sc_gather(table, ids_flat)
    return out.reshape(B, L, D_OUT)


# speedup vs baseline: 5.0348x; 1.3643x over previous
import jax
import jax.numpy as jnp
from jax import lax
from jax.experimental import pallas as pl
from jax.experimental.pallas import tpu as pltpu
from jax.experimental.pallas import tpu_sc as plsc

B, L = 4096, 200
V, D_IN, D_OUT = 100, 32, 16
N = B * L
VPAD = 128

NC, NS = 2, 16
NW = NC * NS
ROWS_PER_W = N // NW
SLAB = 2560
N_SLABS = ROWS_PER_W // SLAB
UNROLL = 16
INNER = SLAB // UNROLL


def _table_kernel(emb_ref, gamma_ref, beta_ref, w_ref, b_ref, out_ref):
    x = emb_ref[...]
    mean = jnp.mean(x, axis=1, keepdims=True)
    xc = x - mean
    var = jnp.mean(xc * xc, axis=1, keepdims=True)
    xn = xc * lax.rsqrt(var + 1e-5)
    xn = xn * gamma_ref[...] + beta_ref[...]
    out_ref[...] = (
        jnp.dot(xn, w_ref[...], preferred_element_type=jnp.float32)
        + b_ref[...]
    )


def _fuse_table(emb_table, ln_gamma, ln_beta, W, b):
    emb_pad = jnp.zeros((VPAD, D_IN), jnp.float32).at[:V].set(emb_table)
    return pl.pallas_call(
        _table_kernel,
        out_shape=jax.ShapeDtypeStruct((VPAD, D_OUT), jnp.float32),
    )(
        emb_pad,
        ln_gamma.reshape(1, D_IN),
        ln_beta.reshape(1, D_IN),
        W,
        b.reshape(1, D_OUT),
    )


def _sc_gather_body(
    table_hbm, ids_hbm, out_hbm, tab_v, idx_v, rows_a, rows_b, sem_a, sem_b
):
    wid = lax.axis_index("s") * NC + lax.axis_index("c")
    row0 = wid * ROWS_PER_W
    pltpu.sync_copy(table_hbm, tab_v)
    pltpu.sync_copy(ids_hbm.at[pl.ds(row0, ROWS_PER_W)], idx_v)

    iota = lax.iota(jnp.int32, 16)

    def make_inner(s):
        def inner(it, carry):
            ids_vec = idx_v[pl.ds(s * SLAB + it * UNROLL, UNROLL)]
            for u in range(UNROLL):
                loc = it * UNROLL + u
                vidx = jnp.full((16,), ids_vec[u] * 16, jnp.int32) + iota
                row = plsc.load_gather(tab_v, [vidx])
                buf = rows_a if s % 2 == 0 else rows_b
                buf[pl.ds(loc * 16, 16)] = row
            return carry

        return inner

    handles = []
    for s in range(N_SLABS):
        if s >= 2:
            handles[s - 2].wait()
        lax.fori_loop(0, INNER, make_inner(s), 0)
        buf = rows_a if s % 2 == 0 else rows_b
        sem = sem_a if s % 2 == 0 else sem_b
        handles.append(
            pltpu.async_copy(
                buf, out_hbm.at[pl.ds((row0 + s * SLAB) * 16, SLAB * 16)], sem
            )
        )
    handles[-2].wait()
    handles[-1].wait()


def _sc_gather(table, ids_flat):
    mesh = plsc.VectorSubcoreMesh(core_axis_name="c", subcore_axis_name="s")
    run = pl.kernel(
        _sc_gather_body,
        out_type=jax.ShapeDtypeStruct((N * D_OUT,), jnp.float32),
        mesh=mesh,
        scratch_types=[
            pltpu.VMEM((VPAD * D_OUT,), jnp.float32),
            pltpu.VMEM((ROWS_PER_W,), jnp.int32),
            pltpu.VMEM((SLAB * D_OUT,), jnp.float32),
            pltpu.VMEM((SLAB * D_OUT,), jnp.float32),
            pltpu.SemaphoreType.DMA,
            pltpu.SemaphoreType.DMA,
        ],
        compiler_params=pltpu.CompilerParams(
            use_tc_tiling_on_sc=False, needs_layout_passes=False
        ),
    )
    return run(table.reshape(VPAD * D_OUT), ids_flat)


def kernel(input_ids, emb_table, ln_gamma, ln_beta, W, b):
    table = _fuse_table(emb_table, ln_gamma, ln_beta, W, b)
    ids_flat = input_ids.reshape(N).astype(jnp.int32)
    out = _sc_gather(table, ids_flat)
    return out.reshape(B, L, D_OUT)


# speedup vs baseline: 5.4333x; 1.0791x over previous
import jax
import jax.numpy as jnp
from jax import lax
from jax.experimental import pallas as pl
from jax.experimental.pallas import tpu as pltpu
from jax.experimental.pallas import tpu_sc as plsc

B, L = 4096, 200
V, D_IN, D_OUT = 100, 32, 16
N = B * L
VPAD = 128

NC, NS = 2, 16
NW = NC * NS
ROWS_PER_W = N // NW
SLAB = 2560
N_SLABS = ROWS_PER_W // SLAB
UNROLL = 16
INNER = SLAB // UNROLL


def _table_kernel(emb_ref, gamma_ref, beta_ref, w_ref, b_ref, out_ref):
    x = emb_ref[...]
    mean = jnp.mean(x, axis=1, keepdims=True)
    xc = x - mean
    var = jnp.mean(xc * xc, axis=1, keepdims=True)
    xn = xc * lax.rsqrt(var + 1e-5)
    xn = xn * gamma_ref[...] + beta_ref[...]
    out_ref[...] = (
        jnp.dot(xn, w_ref[...], preferred_element_type=jnp.float32)
        + b_ref[...]
    )


def _fuse_table(emb_table, ln_gamma, ln_beta, W, b):
    emb_pad = jnp.zeros((VPAD, D_IN), jnp.float32).at[:V].set(emb_table)
    return pl.pallas_call(
        _table_kernel,
        out_shape=jax.ShapeDtypeStruct((VPAD, D_OUT), jnp.float32),
    )(
        emb_pad,
        ln_gamma.reshape(1, D_IN),
        ln_beta.reshape(1, D_IN),
        W,
        b.reshape(1, D_OUT),
    )


def _sc_gather_body(
    table_hbm, ids_hbm, out_hbm, tab_v, idx_v, rows_a, rows_b, sem_a, sem_b
):
    wid = lax.axis_index("s") * NC + lax.axis_index("c")
    row0 = wid * ROWS_PER_W
    pltpu.sync_copy(table_hbm, tab_v)
    pltpu.sync_copy(ids_hbm.at[pl.ds(row0, ROWS_PER_W)], idx_v)

    k16 = lax.iota(jnp.int32, 16) * 16

    def run_inner(s, buf):
        @plsc.parallel_loop(0, SLAB, step=UNROLL)
        def inner(r):
            ids_vec = idx_v[pl.ds(s * SLAB + r, UNROLL)]
            base = ids_vec * 16
            dst = buf.at[pl.ds(r * 16, UNROLL * 16)]
            for c in range(D_OUT):
                col = plsc.load_gather(tab_v, [base + c])
                plsc.store_scatter(dst, [k16 + c], col)

    handles = []
    for s in range(N_SLABS):
        if s >= 2:
            handles[s - 2].wait()
        buf = rows_a if s % 2 == 0 else rows_b
        run_inner(s, buf)
        sem = sem_a if s % 2 == 0 else sem_b
        handles.append(
            pltpu.async_copy(
                buf, out_hbm.at[pl.ds((row0 + s * SLAB) * 16, SLAB * 16)], sem
            )
        )
    handles[-2].wait()
    handles[-1].wait()


def _sc_gather(table, ids_flat):
    mesh = plsc.VectorSubcoreMesh(core_axis_name="c", subcore_axis_name="s")
    run = pl.kernel(
        _sc_gather_body,
        out_type=jax.ShapeDtypeStruct((N * D_OUT,), jnp.float32),
        mesh=mesh,
        scratch_types=[
            pltpu.VMEM((VPAD * D_OUT,), jnp.float32),
            pltpu.VMEM((ROWS_PER_W,), jnp.int32),
            pltpu.VMEM((SLAB * D_OUT,), jnp.float32),
            pltpu.VMEM((SLAB * D_OUT,), jnp.float32),
            pltpu.SemaphoreType.DMA,
            pltpu.SemaphoreType.DMA,
        ],
        compiler_params=pltpu.CompilerParams(
            use_tc_tiling_on_sc=False, needs_layout_passes=False
        ),
    )
    return run(table.reshape(VPAD * D_OUT), ids_flat)


def kernel(input_ids, emb_table, ln_gamma, ln_beta, W, b):
    table = _fuse_table(emb_table, ln_gamma, ln_beta, W, b)
    ids_flat = input_ids.reshape(N).astype(jnp.int32)
    out = _sc_gather(table, ids_flat)
    return out.reshape(B, L, D_OUT)


# speedup vs baseline: 12.7360x; 2.3441x over previous
import jax
import jax.numpy as jnp
from jax import lax
from jax.experimental import pallas as pl
from jax.experimental.pallas import tpu as pltpu
from jax.experimental.pallas import tpu_sc as plsc

B, L = 4096, 200
V, D_IN, D_OUT = 100, 32, 16
N = B * L
VPAD = 128

NC, NS = 2, 16
NW = NC * NS
BT = B // NW
LC = 16
CHUNK_STARTS = [i * LC for i in range(12)] + [L - LC]
OUT_W = 2 * NW * 8 * 128


def _table_kernel(emb_ref, gamma_ref, beta_ref, w_ref, b_ref, out_ref):
    x = emb_ref[...]
    mean = jnp.mean(x, axis=1, keepdims=True)
    xc = x - mean
    var = jnp.mean(xc * xc, axis=1, keepdims=True)
    xn = xc * lax.rsqrt(var + 1e-5)
    xn = xn * gamma_ref[...] + beta_ref[...]
    out_ref[...] = (
        jnp.dot(xn, w_ref[...], preferred_element_type=jnp.float32)
        + b_ref[...]
    )


def _fuse_table(emb_table, ln_gamma, ln_beta, W, b):
    emb_pad = jnp.zeros((VPAD, D_IN), jnp.float32).at[:V].set(emb_table)
    return pl.pallas_call(
        _table_kernel,
        out_shape=jax.ShapeDtypeStruct((VPAD, D_OUT), jnp.float32),
    )(
        emb_pad,
        ln_gamma.reshape(1, D_IN),
        ln_beta.reshape(1, D_IN),
        W,
        b.reshape(1, D_OUT),
    )


def _sc_gather_body(
    table_hbm, ids_hbm, out_hbm,
    tab_v, idx_v, buf00, buf01, buf10, buf11, sem0, sem1,
):
    wid = lax.axis_index("s") * NC + lax.axis_index("c")
    pltpu.sync_copy(table_hbm, tab_v)
    pltpu.sync_copy(ids_hbm.at[pl.ds(wid * (BT * L), BT * L)], idx_v)

    row_iota = lax.iota(jnp.int32, 16)
    bufs = ((buf00, buf01), (buf10, buf11))
    sems = (sem0, sem1)
    handles = []
    for ci, l0 in enumerate(CHUNK_STARTS):
        par = ci % 2
        if ci >= 2:
            handles[2 * (ci - 2)].wait()
            handles[2 * (ci - 2) + 1].wait()
        buf0, buf1 = bufs[par]

        @plsc.parallel_loop(0, BT, step=1)
        def inner(b):
            idv = idx_v[pl.ds(b * L + l0, 16)]
            base = idv * 16
            bc = jnp.full((16,), b, jnp.int32)
            for c8 in range(8):
                colidx = bc + c8 * 128
                for ct, buf in ((0, buf0), (1, buf1)):
                    col = plsc.load_gather(tab_v, [base + (ct * 8 + c8)])
                    plsc.store_scatter(buf, [row_iota, colidx], col)

        for ct, buf in ((0, buf0), (1, buf1)):
            handles.append(
                pltpu.async_copy(
                    buf,
                    out_hbm.at[
                        pl.ds(l0, LC),
                        pl.ds(ct * (NW * 1024) + wid * 1024, 1024),
                    ],
                    sems[par],
                )
            )
    for h in handles[-4:]:
        h.wait()


def _sc_gather(table, ids_flat):
    mesh = plsc.VectorSubcoreMesh(core_axis_name="c", subcore_axis_name="s")
    run = pl.kernel(
        _sc_gather_body,
        out_type=jax.ShapeDtypeStruct((L, OUT_W), jnp.float32),
        mesh=mesh,
        scratch_types=[
            pltpu.VMEM((VPAD * D_OUT,), jnp.float32),
            pltpu.VMEM((BT * L,), jnp.int32),
            pltpu.VMEM((LC, 1024), jnp.float32),
            pltpu.VMEM((LC, 1024), jnp.float32),
            pltpu.VMEM((LC, 1024), jnp.float32),
            pltpu.VMEM((LC, 1024), jnp.float32),
            pltpu.SemaphoreType.DMA,
            pltpu.SemaphoreType.DMA,
        ],
        compiler_params=pltpu.CompilerParams(
            use_tc_tiling_on_sc=False, needs_layout_passes=False
        ),
    )
    return run(table.reshape(VPAD * D_OUT), ids_flat)


def kernel(input_ids, emb_table, ln_gamma, ln_beta, W, b):
    table = _fuse_table(emb_table, ln_gamma, ln_beta, W, b)
    ids_flat = input_ids.reshape(N).astype(jnp.int32)
    out = _sc_gather(table, ids_flat)
    out = out.reshape(L, 2, NW, 8, BT)
    out = out.transpose(2, 4, 0, 1, 3)
    return out.reshape(B, L, D_OUT)
